# baseline (speedup 1.0000x reference)
import functools

import jax
import jax.numpy as jnp
from jax import lax
from jax.experimental import pallas as pl
from jax.experimental.pallas import tpu as pltpu
from jax.experimental.pallas import tpu_sc as plsc

C = 256
N = 10000
E = 160000

NC, NS, L = 2, 16, 16
NW = NC * NS
CHUNK = E // NW
FULL = CHUNK // L

GRID = 10
NBLK = 1024
EBLK = 16384


def _tc_scores(w_ref, z_ref, b_ref, eli_ref, s_ref, src_ref, dst_ref):
    w = w_ref[...]
    w12 = jnp.concatenate([w[:, :C], w[:, C:]], axis=0)
    s = lax.dot_general(
        w12, z_ref[...],
        dimension_numbers=(((1,), (1,)), ((), ())),
        preferred_element_type=jnp.float32,
    )
    row = lax.broadcasted_iota(jnp.int32, (2, NBLK), 0)
    s_ref[...] = s + jnp.where(row == 0, b_ref[0], 0.0)
    src_ref[...] = eli_ref[0]
    dst_ref[...] = eli_ref[1]


_sc_mesh = plsc.VectorSubcoreMesh(core_axis_name="c", subcore_axis_name="s")


@functools.partial(
    pl.kernel,
    out_type=jax.ShapeDtypeStruct((E,), jnp.float32),
    mesh=_sc_mesh,
    scratch_types=[
        pltpu.VMEM((N,), jnp.float32),
        pltpu.VMEM((N,), jnp.float32),
        pltpu.VMEM((CHUNK,), jnp.int32),
        pltpu.VMEM((CHUNK,), jnp.int32),
        pltpu.VMEM((CHUNK,), jnp.float32),
        pltpu.SemaphoreType.DMA,
        pltpu.SemaphoreType.DMA,
        pltpu.SemaphoreType.DMA,
        pltpu.SemaphoreType.DMA,
    ],
    compiler_params=pltpu.CompilerParams(needs_layout_passes=False),
)
def _sc_edge_scores(s_hbm, src_hbm, dst_hbm, out_hbm,
                    s1_v, s2_v, src_v, dst_v, out_v, sem1, sem2, sem3, sem4):
    wid = lax.axis_index("s") * NC + lax.axis_index("c")
    base = wid * CHUNK

    cp1 = pltpu.async_copy(s_hbm.at[0], s1_v, sem1)
    cp2 = pltpu.async_copy(s_hbm.at[1], s2_v, sem2)
    cp3 = pltpu.async_copy(src_hbm.at[pl.ds(base, CHUNK)], src_v, sem3)
    cp4 = pltpu.async_copy(dst_hbm.at[pl.ds(base, CHUNK)], dst_v, sem4)
    cp1.wait()
    cp2.wait()
    cp3.wait()
    cp4.wait()

    @plsc.parallel_loop(0, FULL, unroll=8)
    def _body(i):
        sl = pl.ds(i * L, L)
        out_v[sl] = (plsc.load_gather(s1_v, [src_v[sl]])
                     + plsc.load_gather(s2_v, [dst_v[sl]]))

    tl = pl.ds(CHUNK - L, L)
    out_v[tl] = (plsc.load_gather(s1_v, [src_v[tl]])
                 + plsc.load_gather(s2_v, [dst_v[tl]]))

    pltpu.sync_copy(out_v, out_hbm.at[pl.ds(base, CHUNK)])


def kernel(z, edge_label_index, W, b):
    eli = edge_label_index.astype(jnp.int32)
    s, src, dst = pl.pallas_call(
        _tc_scores,
        grid=(GRID,),
        in_specs=[
            pl.BlockSpec((1, 2 * C), lambda i: (0, 0)),
            pl.BlockSpec((NBLK, C), lambda i: (i, 0)),
            pl.BlockSpec(memory_space=pltpu.SMEM),
            pl.BlockSpec((2, EBLK), lambda i: (0, i)),
        ],
        out_specs=[
            pl.BlockSpec((2, NBLK), lambda i: (0, i)),
            pl.BlockSpec((EBLK,), lambda i: (i,)),
            pl.BlockSpec((EBLK,), lambda i: (i,)),
        ],
        out_shape=[
            jax.ShapeDtypeStruct((2, N), jnp.float32),
            jax.ShapeDtypeStruct((E,), jnp.int32),
            jax.ShapeDtypeStruct((E,), jnp.int32),
        ],
    )(W, z, b, eli)

    return _sc_edge_scores(s, src, dst)


# speedup vs baseline: 1.1142x; 1.1142x over previous
import functools

import jax
import jax.numpy as jnp
from jax import lax
from jax.experimental import pallas as pl
from jax.experimental.pallas import tpu as pltpu
from jax.experimental.pallas import tpu_sc as plsc

C = 256
N = 10000
E = 160000

NC, NS, L = 2, 16, 16
NW = NC * NS
CHUNK = E // NW
FULL = CHUNK // L

GRID = 4
NBLK = 2560
EBLK = 40960


def _tc_scores(w_ref, z_ref, b_ref, eli_ref, s_ref, src_ref, dst_ref):
    w = w_ref[...]
    w12 = jnp.concatenate([w[:, :C], w[:, C:]], axis=0)
    s = lax.dot_general(
        w12, z_ref[...],
        dimension_numbers=(((1,), (1,)), ((), ())),
        preferred_element_type=jnp.float32,
    )
    row = lax.broadcasted_iota(jnp.int32, (2, NBLK), 0)
    s_ref[...] = s + jnp.where(row == 0, b_ref[0], 0.0)
    src_ref[...] = eli_ref[0]
    dst_ref[...] = eli_ref[1]


_sc_mesh = plsc.VectorSubcoreMesh(core_axis_name="c", subcore_axis_name="s")


@functools.partial(
    pl.kernel,
    out_type=jax.ShapeDtypeStruct((E,), jnp.float32),
    mesh=_sc_mesh,
    scratch_types=[
        pltpu.VMEM((N,), jnp.float32),
        pltpu.VMEM((N,), jnp.float32),
        pltpu.VMEM((CHUNK,), jnp.int32),
        pltpu.VMEM((CHUNK,), jnp.int32),
        pltpu.VMEM((CHUNK,), jnp.float32),
        pltpu.SemaphoreType.DMA,
        pltpu.SemaphoreType.DMA,
        pltpu.SemaphoreType.DMA,
        pltpu.SemaphoreType.DMA,
    ],
    compiler_params=pltpu.CompilerParams(needs_layout_passes=False),
)
def _sc_edge_scores(s_hbm, src_hbm, dst_hbm, out_hbm,
                    s1_v, s2_v, src_v, dst_v, out_v, sem1, sem2, sem3, sem4):
    wid = lax.axis_index("s") * NC + lax.axis_index("c")
    base = wid * CHUNK

    cp1 = pltpu.async_copy(s_hbm.at[0], s1_v, sem1)
    cp2 = pltpu.async_copy(s_hbm.at[1], s2_v, sem2)
    cp3 = pltpu.async_copy(src_hbm.at[pl.ds(base, CHUNK)], src_v, sem3)
    cp4 = pltpu.async_copy(dst_hbm.at[pl.ds(base, CHUNK)], dst_v, sem4)
    cp1.wait()
    cp2.wait()
    cp3.wait()
    cp4.wait()

    @plsc.parallel_loop(0, FULL, unroll=8)
    def _body(i):
        sl = pl.ds(i * L, L)
        out_v[sl] = (plsc.load_gather(s1_v, [src_v[sl]])
                     + plsc.load_gather(s2_v, [dst_v[sl]]))

    tl = pl.ds(CHUNK - L, L)
    out_v[tl] = (plsc.load_gather(s1_v, [src_v[tl]])
                 + plsc.load_gather(s2_v, [dst_v[tl]]))

    pltpu.sync_copy(out_v, out_hbm.at[pl.ds(base, CHUNK)])


def kernel(z, edge_label_index, W, b):
    eli = edge_label_index.astype(jnp.int32)
    s, src, dst = pl.pallas_call(
        _tc_scores,
        grid=(GRID,),
        in_specs=[
            pl.BlockSpec((1, 2 * C), lambda i: (0, 0)),
            pl.BlockSpec((NBLK, C), lambda i: (i, 0)),
            pl.BlockSpec(memory_space=pltpu.SMEM),
            pl.BlockSpec((2, EBLK), lambda i: (0, i)),
        ],
        out_specs=[
            pl.BlockSpec((2, NBLK), lambda i: (0, i)),
            pl.BlockSpec((EBLK,), lambda i: (i,)),
            pl.BlockSpec((EBLK,), lambda i: (i,)),
        ],
        out_shape=[
            jax.ShapeDtypeStruct((2, N), jnp.float32),
            jax.ShapeDtypeStruct((E,), jnp.int32),
            jax.ShapeDtypeStruct((E,), jnp.int32),
        ],
    )(W, z, b, eli)

    return _sc_edge_scores(s, src, dst)


# speedup vs baseline: 1.1384x; 1.0217x over previous
import functools

import jax
import jax.numpy as jnp
from jax import lax
from jax.experimental import pallas as pl
from jax.experimental.pallas import tpu as pltpu
from jax.experimental.pallas import tpu_sc as plsc

C = 256
N = 10000
E = 160000

NC, NS, L = 2, 16, 16
NW = NC * NS
CHUNK = E // NW
FULL = CHUNK // L

GRID = 2
NBLK = 5120
EBLK = 81920


def _tc_scores(w_ref, z_ref, b_ref, eli_ref, s_ref, src_ref, dst_ref):
    w = w_ref[...]
    w12 = jnp.concatenate([w[:, :C], w[:, C:]], axis=0)
    s = lax.dot_general(
        w12, z_ref[...],
        dimension_numbers=(((1,), (1,)), ((), ())),
        preferred_element_type=jnp.float32,
    )
    row = lax.broadcasted_iota(jnp.int32, (2, NBLK), 0)
    s_ref[...] = s + jnp.where(row == 0, b_ref[0], 0.0)
    src_ref[...] = eli_ref[0]
    dst_ref[...] = eli_ref[1]


_sc_mesh = plsc.VectorSubcoreMesh(core_axis_name="c", subcore_axis_name="s")


@functools.partial(
    pl.kernel,
    out_type=jax.ShapeDtypeStruct((E,), jnp.float32),
    mesh=_sc_mesh,
    scratch_types=[
        pltpu.VMEM((N,), jnp.float32),
        pltpu.VMEM((N,), jnp.float32),
        pltpu.VMEM((CHUNK,), jnp.int32),
        pltpu.VMEM((CHUNK,), jnp.int32),
        pltpu.VMEM((CHUNK,), jnp.float32),
        pltpu.SemaphoreType.DMA,
        pltpu.SemaphoreType.DMA,
        pltpu.SemaphoreType.DMA,
        pltpu.SemaphoreType.DMA,
    ],
    compiler_params=pltpu.CompilerParams(needs_layout_passes=False),
)
def _sc_edge_scores(s_hbm, src_hbm, dst_hbm, out_hbm,
                    s1_v, s2_v, src_v, dst_v, out_v, sem1, sem2, sem3, sem4):
    wid = lax.axis_index("s") * NC + lax.axis_index("c")
    base = wid * CHUNK

    cp1 = pltpu.async_copy(s_hbm.at[0], s1_v, sem1)
    cp2 = pltpu.async_copy(s_hbm.at[1], s2_v, sem2)
    cp3 = pltpu.async_copy(src_hbm.at[pl.ds(base, CHUNK)], src_v, sem3)
    cp4 = pltpu.async_copy(dst_hbm.at[pl.ds(base, CHUNK)], dst_v, sem4)
    cp1.wait()
    cp2.wait()
    cp3.wait()
    cp4.wait()

    @plsc.parallel_loop(0, FULL, unroll=8)
    def _body(i):
        sl = pl.ds(i * L, L)
        out_v[sl] = (plsc.load_gather(s1_v, [src_v[sl]])
                     + plsc.load_gather(s2_v, [dst_v[sl]]))

    tl = pl.ds(CHUNK - L, L)
    out_v[tl] = (plsc.load_gather(s1_v, [src_v[tl]])
                 + plsc.load_gather(s2_v, [dst_v[tl]]))

    pltpu.sync_copy(out_v, out_hbm.at[pl.ds(base, CHUNK)])


def kernel(z, edge_label_index, W, b):
    eli = edge_label_index.astype(jnp.int32)
    s, src, dst = pl.pallas_call(
        _tc_scores,
        grid=(GRID,),
        in_specs=[
            pl.BlockSpec((1, 2 * C), lambda i: (0, 0)),
            pl.BlockSpec((NBLK, C), lambda i: (i, 0)),
            pl.BlockSpec(memory_space=pltpu.SMEM),
            pl.BlockSpec((2, EBLK), lambda i: (0, i)),
        ],
        out_specs=[
            pl.BlockSpec((2, NBLK), lambda i: (0, i)),
            pl.BlockSpec((EBLK,), lambda i: (i,)),
            pl.BlockSpec((EBLK,), lambda i: (i,)),
        ],
        out_shape=[
            jax.ShapeDtypeStruct((2, N), jnp.float32),
            jax.ShapeDtypeStruct((E,), jnp.int32),
            jax.ShapeDtypeStruct((E,), jnp.int32),
        ],
    )(W, z, b, eli)

    return _sc_edge_scores(s, src, dst)
